# baseline (speedup 1.0000x reference)
import functools

import jax
import jax.numpy as jnp
from jax import lax
from jax.experimental import pallas as pl
from jax.experimental.pallas import tpu as pltpu
from jax.experimental.pallas import tpu_sc as plsc

_N_TOKENS = 32768
_K = 1024
_D = 64
_BT = 128

_NC = 2
_NS = 16
_NW = _NC * _NS
_B_PER_W = _N_TOKENS // _NW


_NL = 128
_NCH = _K // _NL


def _vq_body(z_ref, w_ref, idx_ref):
    z = z_ref[...]
    w = w_ref[...]
    z2 = z + z
    zsq = jnp.sum(z * z, axis=1, keepdims=True)
    wsq = jnp.sum(w * w, axis=1)
    m2 = jax.lax.dot_general(z2, w, (((1,), (1,)), ((), ())))
    lane = jax.lax.broadcasted_iota(jnp.int32, (_BT, _NL), 1)
    bestv = (zsq - m2[:, :_NL]) + wsq[None, :_NL]
    besti = lane
    for j in range(1, _NCH):
        dj = (zsq - m2[:, j * _NL:(j + 1) * _NL]) + wsq[None, j * _NL:(j + 1) * _NL]
        take = dj < bestv
        bestv = jnp.where(take, dj, bestv)
        besti = jnp.where(take, lane + j * _NL, besti)
    minv = jnp.min(bestv, axis=1, keepdims=True)
    cand = jnp.where(bestv == minv, besti, _K)
    idx = jnp.min(cand, axis=1)
    idx_ref[...] = idx[:, None]


def _argmin_indices(z, W):
    nb = _N_TOKENS // _BT
    idx2d = pl.pallas_call(
        _vq_body,
        grid=(nb,),
        in_specs=[
            pl.BlockSpec((_BT, _D), lambda i: (i, 0)),
            pl.BlockSpec((_K, _D), lambda i: (0, 0)),
        ],
        out_specs=pl.BlockSpec((_BT, 1), lambda i: (i, 0)),
        out_shape=jax.ShapeDtypeStruct((_N_TOKENS, 1), jnp.int32),
    )(z, W)
    return idx2d.reshape(_N_TOKENS)


@functools.cache
def _sc_gather_kernel():
    @functools.partial(
        pl.kernel,
        out_type=jax.ShapeDtypeStruct((_N_TOKENS, _D), jnp.float32),
        mesh=plsc.VectorSubcoreMesh(core_axis_name="c", subcore_axis_name="s"),
        scratch_types=[
            pltpu.VMEM((_B_PER_W,), jnp.int32),
            pltpu.VMEM((_B_PER_W, _D), jnp.float32),
            pltpu.SemaphoreType.DMA,
        ],
        compiler_params=pltpu.CompilerParams(use_tc_tiling_on_sc=False),
    )
    def _sc_gather(table_hbm, idx_hbm, out_hbm, idx_v, rows_v, sem):
        wid = lax.axis_index("s") * _NC + lax.axis_index("c")
        base = wid * _B_PER_W
        pltpu.sync_copy(idx_hbm.at[pl.ds(base, _B_PER_W)], idx_v)
        pltpu.async_copy(table_hbm.at[idx_v], rows_v, sem).wait()
        pltpu.sync_copy(rows_v, out_hbm.at[pl.ds(base, _B_PER_W)])

    return _sc_gather


def kernel(z, W):
    idx = _argmin_indices(z, W)
    zq = _sc_gather_kernel()(W, idx)
    return (zq, idx)


# speedup vs baseline: 1.5830x; 1.5830x over previous
import functools

import jax
import jax.numpy as jnp
from jax import lax
from jax.experimental import pallas as pl
from jax.experimental.pallas import tpu as pltpu
from jax.experimental.pallas import tpu_sc as plsc

_N_TOKENS = 32768
_K = 1024
_D = 64
_BT = 512

_NC = 2
_NS = 16
_NW = _NC * _NS
_B_PER_W = _N_TOKENS // _NW
_CHUNK = _B_PER_W // 2

_SUB = 128
_NL = 128
_NCH = _K // _NL


def _vq_body(z_ref, w_ref, idx_ref):
    z = z_ref[...]
    w = w_ref[...]
    z2 = z + z
    zsq = jnp.sum(z * z, axis=1, keepdims=True)
    wsq = jnp.sum(w * w, axis=1)
    m2 = jax.lax.dot_general(z2, w, (((1,), (1,)), ((), ())))
    lane = jax.lax.broadcasted_iota(jnp.int32, (_SUB, _NL), 1)
    for s in range(_BT // _SUB):
        rows = slice(s * _SUB, (s + 1) * _SUB)
        zsq_s = zsq[rows, :]
        bestv = (zsq_s - m2[rows, :_NL]) + wsq[None, :_NL]
        besti = lane
        for j in range(1, _NCH):
            dj = (zsq_s - m2[rows, j * _NL:(j + 1) * _NL]) + wsq[None, j * _NL:(j + 1) * _NL]
            take = dj < bestv
            bestv = jnp.where(take, dj, bestv)
            besti = jnp.where(take, lane + j * _NL, besti)
        minv = jnp.min(bestv, axis=1, keepdims=True)
        cand = jnp.where(bestv == minv, besti, _K)
        idx = jnp.min(cand, axis=1)
        idx_ref[pl.ds(s * _SUB, _SUB)] = idx


def _argmin_indices(z, W):
    nb = _N_TOKENS // _BT
    return pl.pallas_call(
        _vq_body,
        grid=(nb,),
        in_specs=[
            pl.BlockSpec((_BT, _D), lambda i: (i, 0)),
            pl.BlockSpec((_K, _D), lambda i: (0, 0)),
        ],
        out_specs=pl.BlockSpec((_BT,), lambda i: (i,)),
        out_shape=jax.ShapeDtypeStruct((_N_TOKENS,), jnp.int32),
    )(z, W)


@functools.cache
def _sc_gather_kernel():
    @functools.partial(
        pl.kernel,
        out_type=jax.ShapeDtypeStruct((_N_TOKENS, _D), jnp.float32),
        mesh=plsc.VectorSubcoreMesh(core_axis_name="c", subcore_axis_name="s"),
        scratch_types=[
            pltpu.VMEM((_B_PER_W,), jnp.int32),
            pltpu.VMEM((_CHUNK, 2 * _D), jnp.float32),
            pltpu.SemaphoreType.DMA,
        ],
        compiler_params=pltpu.CompilerParams(use_tc_tiling_on_sc=False),
    )
    def _sc_gather(table_hbm, idx_hbm, out_hbm, idx_v, rows_v, sem):
        wid = lax.axis_index("s") * _NC + lax.axis_index("c")
        base = wid * _B_PER_W
        pltpu.sync_copy(idx_hbm.at[pl.ds(base, _B_PER_W)], idx_v)
        for c in range(_B_PER_W // _CHUNK):
            pltpu.async_copy(
                table_hbm.at[idx_v.at[pl.ds(c * _CHUNK, _CHUNK)]], rows_v, sem
            ).wait()
            pltpu.sync_copy(
                rows_v.at[:, pl.ds(0, _D)],
                out_hbm.at[pl.ds(base + c * _CHUNK, _CHUNK)],
            )

    return _sc_gather


def kernel(z, W):
    idx = _argmin_indices(z, W)
    table = jnp.pad(W, ((0, 0), (0, 2 * _D - _D)))
    zq = _sc_gather_kernel()(table, idx)
    return (zq, idx)
